# baseline (speedup 1.0000x reference)
import functools

import jax
import jax.numpy as jnp
from jax import lax
from jax.experimental import pallas as pl
from jax.experimental.pallas import tpu as pltpu
from jax.experimental.pallas import tpu_sc as plsc

B = 16384
L = 50
D = 64
V = 1000000
VP = V // 2
TP = 128
LANES = 16
NC, NS = 2, 16
NW = NC * NS
RW = B // NW
CB = 8
NCHUNK = RW // CB
IDXW = 80
NSUB = CB * L // IDXW
NROWIDX = RW * L // IDXW
DCH = D // LANES

_mesh = plsc.VectorSubcoreMesh(core_axis_name="c", subcore_axis_name="s")


@functools.partial(
    pl.kernel,
    mesh=_mesh,
    out_type=jax.ShapeDtypeStruct((B * D,), jnp.float32),
    compiler_params=pltpu.CompilerParams(use_tc_tiling_on_sc=True),
    scratch_types=[
        pltpu.VMEM((NROWIDX, TP), jnp.int32),
        pltpu.VMEM((CB * L, TP), jnp.float32),
        pltpu.VMEM((RW + CB,), jnp.int32),
        pltpu.VMEM((RW + CB,), jnp.int32),
        pltpu.VMEM((RW * D,), jnp.float32),
        pltpu.SemaphoreType.DMA,
    ],
)
def _pooled_embed(x_hbm, len_hbm, ne_hbm, table_hbm, out_hbm,
                  idx_v, rows_v, len_v, ne_v, out_v, sem):
    wid = lax.axis_index("s") * NC + lax.axis_index("c")
    base_row = wid * RW
    pltpu.sync_copy(len_hbm.at[pl.ds(base_row, RW)], len_v.at[pl.ds(0, RW)])
    pltpu.sync_copy(ne_hbm.at[pl.ds(base_row, RW)], ne_v.at[pl.ds(0, RW)])
    pltpu.sync_copy(x_hbm.at[pl.ds(wid * NROWIDX, NROWIDX), :], idx_v)

    @pl.loop(0, NCHUNK)
    def chunk_body(ci):
        copies = [
            pltpu.async_copy(
                table_hbm.at[idx_v.at[ci * NSUB + k, pl.ds(0, IDXW)]],
                rows_v.at[pl.ds(k * IDXW, IDXW)],
                sem,
            )
            for k in range(NSUB)
        ]
        for cp in copies:
            cp.wait()

        ln_vec = len_v[pl.ds(ci * CB, LANES)]
        ne_vec = ne_v[pl.ds(ci * CB, LANES)]
        for r in range(CB):
            lnf = jnp.broadcast_to(ln_vec[r], (LANES,)).astype(jnp.float32)
            inv = 1.0 / lnf
            ne = ne_vec[r]

            def tok_body(j, accs):
                off = jnp.where(j < ne, 0, D)
                return tuple(
                    accs[c]
                    + rows_v[r * L + j, pl.ds(off + c * LANES, LANES)]
                    for c in range(DCH)
                )

            zeros = tuple(jnp.zeros((LANES,), jnp.float32) for _ in range(DCH))
            accs = lax.fori_loop(0, L, tok_body, zeros, unroll=2)
            obase = (ci * CB + r) * D
            for c in range(DCH):
                out_v[pl.ds(obase + c * LANES, LANES)] = accs[c] * inv

    pltpu.sync_copy(out_v, out_hbm.at[pl.ds(base_row * D, RW * D)])


def kernel(x, tensor_lengths, table):
    xi = x.astype(jnp.int32)
    ln = tensor_lengths.astype(jnp.int32)
    valid = jnp.arange(L, dtype=jnp.int32)[None, :] < ln[:, None]
    par = xi & 1
    key = jnp.where(valid, par, 2)
    order = jnp.argsort(key, axis=1, stable=True)
    xs = jnp.take_along_axis(xi, order, axis=1)
    vs = jnp.take_along_axis(valid, order, axis=1)
    xp = jnp.where(vs, xs >> 1, VP)
    ne = jnp.sum(jnp.where(valid, 1 - par, 0), axis=1, dtype=jnp.int32)

    x2 = xp.reshape(B * L // IDXW, IDXW)
    x3 = jnp.pad(x2, ((0, 0), (0, TP - IDXW)))
    tp = jnp.concatenate(
        [table.reshape(VP, TP), jnp.zeros((8, TP), jnp.float32)]
    )
    out = _pooled_embed(x3, ln, ne, tp)
    return out.reshape(B, D)


# speedup vs baseline: 22.3372x; 22.3372x over previous
import functools

import jax
import jax.numpy as jnp
from jax import lax
from jax.experimental import pallas as pl
from jax.experimental.pallas import tpu as pltpu
from jax.experimental.pallas import tpu_sc as plsc

B = 16384
L = 50
D = 64
V = 1000000
LANES = 16
NC, NS = 2, 16
NW = NC * NS
RW = B // NW
CB = 8
NCHUNK = RW // CB
IDXW = 80
NSUB = CB * L // IDXW
NROWIDX = RW * L // IDXW
DCH = D // LANES

_mesh = plsc.VectorSubcoreMesh(core_axis_name="c", subcore_axis_name="s")


@functools.partial(
    pl.kernel,
    mesh=_mesh,
    out_type=jax.ShapeDtypeStruct((B * D,), jnp.float32),
    compiler_params=pltpu.CompilerParams(use_tc_tiling_on_sc=False),
    scratch_types=[
        pltpu.VMEM((NROWIDX, IDXW), jnp.int32),
        pltpu.VMEM((2, CB * L, D), jnp.float32),
        pltpu.VMEM((RW + CB,), jnp.int32),
        pltpu.VMEM((RW * D,), jnp.float32),
        pltpu.SemaphoreType.DMA,
        pltpu.SemaphoreType.DMA,
    ],
)
def _pooled_embed(x_hbm, len_hbm, table_hbm, out_hbm,
                  idx_v, rows_v, len_v, out_v, sem0, sem1):
    wid = lax.axis_index("s") * NC + lax.axis_index("c")
    base_row = wid * RW
    sems = (sem0, sem1)
    pltpu.sync_copy(len_hbm.at[pl.ds(base_row, RW)], len_v.at[pl.ds(0, RW)])
    pltpu.sync_copy(x_hbm.at[pl.ds(wid * NROWIDX, NROWIDX), :], idx_v)

    def fire(ci, b, sem):
        for k in range(NSUB):
            pltpu.async_copy(
                table_hbm.at[idx_v.at[ci * NSUB + k]],
                rows_v.at[b, pl.ds(k * IDXW, IDXW)],
                sem,
            )

    def drain(ci, b, sem):
        for k in range(NSUB):
            pltpu.make_async_copy(
                table_hbm.at[idx_v.at[ci * NSUB + k]],
                rows_v.at[b, pl.ds(k * IDXW, IDXW)],
                sem,
            ).wait()

    def compute(ci, b):
        ln_vec = len_v[pl.ds(ci * CB, LANES)]
        for r in range(CB):
            lnv = jnp.broadcast_to(ln_vec[r], (LANES,))
            lnf = lnv.astype(jnp.float32)
            inv = 1.0 / lnf

            def tok_body(j, accs):
                jv = jnp.broadcast_to(j, (LANES,))
                mf = jnp.where(jv < lnv, 1.0, 0.0).astype(jnp.float32)
                return tuple(
                    accs[c]
                    + rows_v[b, r * L + j, pl.ds(c * LANES, LANES)] * mf
                    for c in range(DCH)
                )

            zeros = tuple(jnp.zeros((LANES,), jnp.float32) for _ in range(DCH))
            accs = lax.fori_loop(0, L, tok_body, zeros, unroll=2)
            obase = (ci * CB + r) * D
            for c in range(DCH):
                out_v[pl.ds(obase + c * LANES, LANES)] = accs[c] * inv

    fire(0, 0, sems[0])

    @pl.loop(0, NCHUNK, step=2)
    def chunk_body(ci):
        for b in range(2):
            cur = ci + b

            @pl.when(cur + 1 < NCHUNK)
            def _():
                fire(cur + 1, 1 - b, sems[1 - b])

            drain(cur, b, sems[b])
            compute(cur, b)

    pltpu.sync_copy(out_v, out_hbm.at[pl.ds(base_row * D, RW * D)])


def kernel(x, tensor_lengths, table):
    x2 = x.astype(jnp.int32).reshape(B * L // IDXW, IDXW)
    ln = tensor_lengths.astype(jnp.int32)
    out = _pooled_embed(x2, ln, table)
    return out.reshape(B, D)
